# baseline (speedup 1.0000x reference)
import functools

import jax
import jax.numpy as jnp
from jax import lax
from jax.experimental import pallas as pl
from jax.experimental.pallas import tpu as pltpu
from jax.experimental.pallas import tpu_sc as plsc

_ROW = 32
_CHUNK = 128


@functools.cache
def _sc_build(num_rows: int, nidx: int):
  info = plsc.get_sparse_core_info()
  nc, ns = info.num_cores, info.num_subcores
  nw = nc * ns
  b_per_w = nidx // nw
  n_chunks = b_per_w // _CHUNK
  mesh = plsc.VectorSubcoreMesh(core_axis_name="c", subcore_axis_name="s")

  @functools.partial(
      pl.kernel,
      mesh=mesh,
      out_type=jax.ShapeDtypeStruct((nw, n_chunks, _CHUNK, _ROW),
                                    jnp.float32),
      scratch_types=[
          pltpu.VMEM((n_chunks, _CHUNK), jnp.int32),
          pltpu.VMEM((n_chunks, _CHUNK, _ROW), jnp.float32),
          pltpu.SemaphoreType.DMA,
      ],
      compiler_params=pltpu.CompilerParams(use_tc_tiling_on_sc=False),
  )
  def gather(idx_hbm, tbl_hbm, out_hbm, idx_v, rows_v, sem):
    wid = lax.axis_index("s") * nc + lax.axis_index("c")
    base = wid * b_per_w
    for j in range(n_chunks):
      pltpu.sync_copy(idx_hbm.at[pl.ds(base + j * _CHUNK, _CHUNK)],
                      idx_v.at[j])
    copies = [
        pltpu.async_copy(tbl_hbm.at[idx_v.at[j]], rows_v.at[j], sem)
        for j in range(n_chunks)
    ]
    for c in copies:
      c.wait()
    pltpu.sync_copy(rows_v, out_hbm.at[wid])

  return gather


def kernel(ids, boxes):
  num_models, num_boxes, two, dim = boxes.shape
  batch = ids.shape[0]
  ids32 = ids.astype(jnp.int32)
  idx_all = jnp.concatenate([ids32, ids32 + num_boxes])
  tbl = boxes.reshape(num_models * num_boxes, _ROW)
  out = _sc_build(num_models * num_boxes, num_models * batch)(idx_all, tbl)
  return out.reshape(num_models, batch, two, dim)


# speedup vs baseline: 54.5371x; 54.5371x over previous
import functools

import jax
import jax.numpy as jnp
from jax import lax
from jax.experimental import pallas as pl
from jax.experimental.pallas import tpu as pltpu
from jax.experimental.pallas import tpu_sc as plsc

_NBUF = 4
_L = 16


@functools.cache
def _build(num_models: int, num_boxes: int, batch: int, dim: int):
  info = plsc.get_sparse_core_info()
  nc, ns = info.num_cores, info.num_subcores
  nw = nc * ns
  b_per_w = batch // nw
  ngrp = b_per_w // _NBUF
  row_w = num_models * 2 * dim
  stg_rows = b_per_w * row_w // 128
  mesh = plsc.VectorSubcoreMesh(core_axis_name="c", subcore_axis_name="s")

  @functools.partial(
      pl.kernel,
      mesh=mesh,
      out_type=jax.ShapeDtypeStruct((nw, stg_rows, 128), jnp.float32),
      scratch_types=[
          pltpu.VMEM((b_per_w + _L,), jnp.int32),
          pltpu.VMEM((_NBUF, num_models, 2, dim, 128), jnp.float32),
          pltpu.VMEM((stg_rows, 128), jnp.float32),
          pltpu.SemaphoreType.DMA,
      ],
      compiler_params=pltpu.CompilerParams(use_tc_tiling_on_sc=True,
                                           needs_layout_passes=False),
  )
  def gather(ids_hbm, bt_hbm, out_hbm, ids_v, buf_v, stg_v, sem):
    wid = lax.axis_index("s") * nc + lax.axis_index("c")
    base = wid * b_per_w
    pltpu.sync_copy(ids_hbm.at[pl.ds(base, b_per_w)],
                    ids_v.at[pl.ds(0, b_per_w)])

    def idat(j):
      return ids_v[pl.ds(j, _L)][0]

    def issue(j, slot):
      idv = idat(j)
      off = pl.multiple_of((idv >> 7) << 7, 128)
      pltpu.async_copy(bt_hbm.at[:, :, :, pl.ds(off, 128)],
                       buf_v.at[slot], sem)

    for b in range(_NBUF):
      issue(b, b)

    lane_iota = lax.iota(jnp.int32, _L)

    def group(g, carry):
      for b in range(_NBUF):
        j = g * _NBUF + b
        pltpu.make_async_copy(bt_hbm.at[:, :, :, pl.ds(0, 128)],
                              buf_v.at[b], sem).wait()
        idv = idat(j)
        col = jnp.full((_L,), idv & 127, jnp.int32)
        r = j // 2
        cbase = (j % 2) * row_w
        for m in range(num_models):
          for c in range(2):
            vals = plsc.load_gather(buf_v.at[b, m, c], [lane_iota, col])
            stg_v[r, pl.ds(cbase + (m * 2 + c) * dim, _L)] = vals

        @pl.when(g + 1 < ngrp)
        def _():
          issue(j + _NBUF, b)

      return carry

    lax.fori_loop(0, ngrp, group, 0)
    pltpu.sync_copy(stg_v, out_hbm.at[wid])

  return gather


def kernel(ids, boxes):
  num_models, num_boxes, two, dim = boxes.shape
  batch = ids.shape[0]
  bt = boxes.transpose(0, 2, 3, 1)
  out = _build(num_models, num_boxes, batch, dim)(ids.astype(jnp.int32), bt)
  nw = out.shape[0]
  out = out.reshape(nw, batch // nw, num_models, two, dim)
  return out.transpose(2, 0, 1, 3, 4).reshape(num_models, batch, two, dim)


# speedup vs baseline: 56.6409x; 1.0386x over previous
import functools

import jax
import jax.numpy as jnp
from jax import lax
from jax.experimental import pallas as pl
from jax.experimental.pallas import tpu as pltpu
from jax.experimental.pallas import tpu_sc as plsc

_NBUF = 8
_L = 16


@functools.cache
def _build(num_models: int, num_boxes: int, batch: int, dim: int):
  info = plsc.get_sparse_core_info()
  nc, ns = info.num_cores, info.num_subcores
  nw = nc * ns
  b_per_w = batch // nw
  ngrp = b_per_w // _NBUF
  row_w = num_models * 2 * dim
  stg_rows = b_per_w * row_w // 128
  mesh = plsc.VectorSubcoreMesh(core_axis_name="c", subcore_axis_name="s")

  @functools.partial(
      pl.kernel,
      mesh=mesh,
      out_type=jax.ShapeDtypeStruct((nw, stg_rows, 128), jnp.float32),
      scratch_types=[
          pltpu.VMEM((b_per_w + _L,), jnp.int32),
          pltpu.VMEM((_NBUF, num_models, 2, dim, 128), jnp.float32),
          pltpu.VMEM((stg_rows, 128), jnp.float32),
          pltpu.SemaphoreType.DMA,
      ],
      compiler_params=pltpu.CompilerParams(use_tc_tiling_on_sc=True,
                                           needs_layout_passes=False),
  )
  def gather(ids_hbm, bt_hbm, out_hbm, ids_v, buf_v, stg_v, sem):
    wid = lax.axis_index("s") * nc + lax.axis_index("c")
    base = wid * b_per_w
    pltpu.sync_copy(ids_hbm.at[pl.ds(base, b_per_w)],
                    ids_v.at[pl.ds(0, b_per_w)])

    def idat(j):
      return ids_v[pl.ds(j, _L)][0]

    def issue(j, slot):
      idv = idat(j)
      off = pl.multiple_of((idv >> 7) << 7, 128)
      pltpu.async_copy(bt_hbm.at[:, :, :, pl.ds(off, 128)],
                       buf_v.at[slot], sem)

    for b in range(_NBUF):
      issue(b, b)

    lane_iota = lax.iota(jnp.int32, _L)

    def group(g, carry):
      for b in range(_NBUF):
        j = g * _NBUF + b
        pltpu.make_async_copy(bt_hbm.at[:, :, :, pl.ds(0, 128)],
                              buf_v.at[b], sem).wait()
        idv = idat(j)
        col = jnp.full((_L,), idv & 127, jnp.int32)
        r = j // 2
        cbase = (j % 2) * row_w
        for m in range(num_models):
          for c in range(2):
            vals = plsc.load_gather(buf_v.at[b, m, c], [lane_iota, col])
            stg_v[r, pl.ds(cbase + (m * 2 + c) * dim, _L)] = vals

        @pl.when(g + 1 < ngrp)
        def _():
          issue(j + _NBUF, b)

      return carry

    lax.fori_loop(0, ngrp, group, 0)
    pltpu.sync_copy(stg_v, out_hbm.at[wid])

  return gather


def kernel(ids, boxes):
  num_models, num_boxes, two, dim = boxes.shape
  batch = ids.shape[0]
  bt = boxes.transpose(0, 2, 3, 1)
  out = _build(num_models, num_boxes, batch, dim)(ids.astype(jnp.int32), bt)
  nw = out.shape[0]
  out = out.reshape(nw, batch // nw, num_models, two, dim)
  return out.transpose(2, 0, 1, 3, 4).reshape(num_models, batch, two, dim)
